# baseline (speedup 1.0000x reference)
import jax
import jax.numpy as jnp
from jax import lax
from jax.experimental import pallas as pl
from jax.experimental.pallas import tpu as pltpu

B = 16384
D = 1024
C = 1000
CP = 1024
M = 2 * CP
NPAD = 2 * (CP - C)

RB1 = 1024
NB1 = B // RB1
RB2 = 2048
NB2 = B // RB2

F8S = 16.0
F8 = jnp.float8_e4m3fn


def _stats_body(feat_ref, lblp_ref, mem_ref, src_ref,
                featn_ref, memot_ref, lltot_ref,
                sums_ref, pres_ref, xprev_ref):
    i = pl.program_id(0)

    lblp = lblp_ref[0, 0, :]
    cls = lax.broadcasted_iota(jnp.int32, (CP, RB1), 0)
    eq = cls == lblp[None, :]
    d = lax.dot_general(
        eq.astype(F8), xprev_ref[...],
        (((1,), (0,)), ((), ())), preferred_element_type=jnp.float32)
    pm = jnp.any(eq, axis=1, keepdims=True).astype(jnp.float32)
    sums_ref[...] = jnp.where(i > 0, sums_ref[...] + d, 0.0)
    pres_ref[...] = jnp.where(i > 0, jnp.maximum(pres_ref[...], pm), 0.0)

    x = feat_ref[...]
    ss = jnp.sum(x * x, axis=1, keepdims=True)
    inv = F8S / jnp.maximum(jnp.sqrt(ss), 1e-12)
    xf8 = (x * inv).astype(F8)
    featn_ref[...] = xf8
    xprev_ref[...] = xf8

    @pl.when(i == NB1)
    def _():
        sums = sums_ref[0:C, :]
        present = pres_ref[0:C, :] > 0.0
        snorm = jnp.sqrt(jnp.sum(sums * sums, axis=1, keepdims=True))
        bc = jnp.where(present, sums / jnp.maximum(snorm, 1e-12), 0.0)
        mem = mem_ref[...]
        uw = jnp.sum(mem * bc, axis=1, keepdims=True)
        uw = 1.0 - (1.0 - uw) * present.astype(jnp.float32)
        nm = uw * mem + (1.0 - uw) * bc
        nnorm = jnp.sqrt(jnp.sum(nm * nm, axis=1, keepdims=True))
        nm = nm / jnp.maximum(nnorm, 1e-12)
        zpad = jnp.zeros((CP - C, D), jnp.float32)
        nmp = jnp.concatenate([nm * F8S, zpad], axis=0)
        srcp = jnp.concatenate([src_ref[...] * F8S, zpad], axis=0)
        memot_ref[:, 0:CP] = jnp.transpose(nmp).astype(F8)
        memot_ref[:, CP:M] = jnp.transpose(srcp).astype(F8)
        lltot_ref[...] = (jnp.sum(sums * nm) / F8S).reshape(1, 1)


def _loss_body(featn_ref, memot_ref, lltot_ref, out_ref, lprev_ref, acc_ref):
    i = pl.program_id(0)

    e = jnp.exp2(lprev_ref[...])
    es = jnp.sum(e, axis=1, keepdims=True).astype(jnp.float32)
    lse = jnp.log(es - float(NPAD))
    acc_ref[...] = jnp.where(i > 0, acc_ref[...] + lse, 0.0)

    raw = lax.dot_general(
        featn_ref[...], memot_ref[...],
        (((1,), (0,)), ((), ())), preferred_element_type=jnp.float32)
    lprev_ref[...] = (raw * (1.4426950408889634 / (F8S * F8S))
                      ).astype(jnp.bfloat16)

    @pl.when(i == NB2)
    def _():
        out_ref[...] = (jnp.sum(acc_ref[...]).reshape(1, 1)
                        - lltot_ref[...]) / float(B)


@jax.jit
def kernel(feat, label, memory, source_memo):
    lbl3 = label.astype(jnp.int32).reshape(NB1, 1, RB1)

    featn, memot, lltot = pl.pallas_call(
        _stats_body,
        grid=(NB1 + 1,),
        in_specs=[
            pl.BlockSpec((RB1, D), lambda i: (jnp.minimum(i, NB1 - 1), 0)),
            pl.BlockSpec((1, 1, RB1), lambda i: (jnp.maximum(i - 1, 0), 0, 0)),
            pl.BlockSpec((C, D), lambda i: (0, 0)),
            pl.BlockSpec((C, D), lambda i: (0, 0)),
        ],
        out_specs=[
            pl.BlockSpec((RB1, D), lambda i: (jnp.minimum(i, NB1 - 1), 0)),
            pl.BlockSpec((D, M), lambda i: (0, 0)),
            pl.BlockSpec((1, 1), lambda i: (0, 0)),
        ],
        out_shape=[
            jax.ShapeDtypeStruct((B, D), F8),
            jax.ShapeDtypeStruct((D, M), F8),
            jax.ShapeDtypeStruct((1, 1), jnp.float32),
        ],
        scratch_shapes=[
            pltpu.VMEM((CP, D), jnp.float32),
            pltpu.VMEM((CP, 1), jnp.float32),
            pltpu.VMEM((RB1, D), F8),
        ],
        compiler_params=pltpu.CompilerParams(
            dimension_semantics=("arbitrary",)),
    )(feat, lbl3, memory, source_memo)

    loss2d = pl.pallas_call(
        _loss_body,
        grid=(NB2 + 1,),
        in_specs=[
            pl.BlockSpec((RB2, D), lambda i: (jnp.minimum(i, NB2 - 1), 0)),
            pl.BlockSpec((D, M), lambda i: (0, 0)),
            pl.BlockSpec((1, 1), lambda i: (0, 0)),
        ],
        out_specs=pl.BlockSpec((1, 1), lambda i: (0, 0)),
        out_shape=jax.ShapeDtypeStruct((1, 1), jnp.float32),
        scratch_shapes=[
            pltpu.VMEM((RB2, M), jnp.bfloat16),
            pltpu.VMEM((RB2, 1), jnp.float32),
        ],
        compiler_params=pltpu.CompilerParams(
            dimension_semantics=("arbitrary",)),
    )(featn, memot, lltot)

    return loss2d[0, 0]


# speedup vs baseline: 1.0822x; 1.0822x over previous
import jax
import jax.numpy as jnp
from jax import lax
from jax.experimental import pallas as pl
from jax.experimental.pallas import tpu as pltpu

B = 16384
D = 1024
C = 1000
CP = 1024
M = 2 * CP
NPAD = 2 * (CP - C)

RB1 = 1024
NB1 = B // RB1
RB2 = 1024
NB2 = B // RB2

F8S = 16.0
F8 = jnp.float8_e4m3fn


def _stats_body(feat_ref, lblp_ref, mem_ref, src_ref,
                featn_ref, memot_ref, lltot_ref,
                sums_ref, pres_ref, xprev_ref):
    i = pl.program_id(0)

    lblp = lblp_ref[0, 0, :]
    cls = lax.broadcasted_iota(jnp.int32, (CP, RB1), 0)
    eq = cls == lblp[None, :]
    d = lax.dot_general(
        eq.astype(F8), xprev_ref[...],
        (((1,), (0,)), ((), ())), preferred_element_type=jnp.float32)
    pm = jnp.any(eq, axis=1, keepdims=True).astype(jnp.float32)
    sums_ref[...] = jnp.where(i > 0, sums_ref[...] + d, 0.0)
    pres_ref[...] = jnp.where(i > 0, jnp.maximum(pres_ref[...], pm), 0.0)

    x = feat_ref[...]
    ss = jnp.sum(x * x, axis=1, keepdims=True)
    inv = F8S / jnp.maximum(jnp.sqrt(ss), 1e-12)
    xf8 = (x * inv).astype(F8)
    featn_ref[...] = xf8
    xprev_ref[...] = xf8

    @pl.when(i == NB1)
    def _():
        sums = sums_ref[0:C, :]
        present = pres_ref[0:C, :] > 0.0
        snorm = jnp.sqrt(jnp.sum(sums * sums, axis=1, keepdims=True))
        bc = jnp.where(present, sums / jnp.maximum(snorm, 1e-12), 0.0)
        mem = mem_ref[...]
        uw = jnp.sum(mem * bc, axis=1, keepdims=True)
        uw = 1.0 - (1.0 - uw) * present.astype(jnp.float32)
        nm = uw * mem + (1.0 - uw) * bc
        nnorm = jnp.sqrt(jnp.sum(nm * nm, axis=1, keepdims=True))
        nm = nm / jnp.maximum(nnorm, 1e-12)
        zpad = jnp.zeros((CP - C, D), jnp.float32)
        nmp = jnp.concatenate([nm * F8S, zpad], axis=0)
        srcp = jnp.concatenate([src_ref[...] * F8S, zpad], axis=0)
        memot_ref[:, 0:CP] = jnp.transpose(nmp).astype(F8)
        memot_ref[:, CP:M] = jnp.transpose(srcp).astype(F8)
        lltot_ref[...] = (jnp.sum(sums * nm) / F8S).reshape(1, 1)


def _loss_body(featn_ref, memot_ref, lltot_ref, out_ref, lprev_ref, acc_ref):
    i = pl.program_id(0)

    e = jnp.exp2(lprev_ref[...])
    es = jnp.sum(e, axis=1, keepdims=True).astype(jnp.float32)
    lse = jnp.log(es - float(NPAD))
    acc_ref[...] = jnp.where(i > 0, acc_ref[...] + lse, 0.0)

    raw = lax.dot_general(
        featn_ref[...], memot_ref[...],
        (((1,), (0,)), ((), ())), preferred_element_type=jnp.float32)
    lprev_ref[...] = (raw * (1.4426950408889634 / (F8S * F8S))
                      ).astype(jnp.bfloat16)

    @pl.when(i == NB2)
    def _():
        out_ref[...] = (jnp.sum(acc_ref[...]).reshape(1, 1)
                        - lltot_ref[...]) / float(B)


@jax.jit
def kernel(feat, label, memory, source_memo):
    lbl3 = label.astype(jnp.int32).reshape(NB1, 1, RB1)

    featn, memot, lltot = pl.pallas_call(
        _stats_body,
        grid=(NB1 + 1,),
        in_specs=[
            pl.BlockSpec((RB1, D), lambda i: (jnp.minimum(i, NB1 - 1), 0)),
            pl.BlockSpec((1, 1, RB1), lambda i: (jnp.maximum(i - 1, 0), 0, 0)),
            pl.BlockSpec((C, D), lambda i: (0, 0)),
            pl.BlockSpec((C, D), lambda i: (0, 0)),
        ],
        out_specs=[
            pl.BlockSpec((RB1, D), lambda i: (jnp.minimum(i, NB1 - 1), 0)),
            pl.BlockSpec((D, M), lambda i: (0, 0)),
            pl.BlockSpec((1, 1), lambda i: (0, 0)),
        ],
        out_shape=[
            jax.ShapeDtypeStruct((B, D), F8),
            jax.ShapeDtypeStruct((D, M), F8),
            jax.ShapeDtypeStruct((1, 1), jnp.float32),
        ],
        scratch_shapes=[
            pltpu.VMEM((CP, D), jnp.float32),
            pltpu.VMEM((CP, 1), jnp.float32),
            pltpu.VMEM((RB1, D), F8),
        ],
        compiler_params=pltpu.CompilerParams(
            dimension_semantics=("arbitrary",)),
    )(feat, lbl3, memory, source_memo)

    loss2d = pl.pallas_call(
        _loss_body,
        grid=(NB2 + 1,),
        in_specs=[
            pl.BlockSpec((RB2, D), lambda i: (jnp.minimum(i, NB2 - 1), 0)),
            pl.BlockSpec((D, M), lambda i: (0, 0)),
            pl.BlockSpec((1, 1), lambda i: (0, 0)),
        ],
        out_specs=pl.BlockSpec((1, 1), lambda i: (0, 0)),
        out_shape=jax.ShapeDtypeStruct((1, 1), jnp.float32),
        scratch_shapes=[
            pltpu.VMEM((RB2, M), jnp.bfloat16),
            pltpu.VMEM((RB2, 1), jnp.float32),
        ],
        compiler_params=pltpu.CompilerParams(
            dimension_semantics=("arbitrary",)),
    )(featn, memot, lltot)

    return loss2d[0, 0]


# speedup vs baseline: 1.1792x; 1.0896x over previous
import jax
import jax.numpy as jnp
from jax import lax
from jax.experimental import pallas as pl
from jax.experimental.pallas import tpu as pltpu

B = 16384
D = 1024
C = 1000
CP = 1024
M = 2 * CP
NPAD = 2 * (CP - C)

RB1 = 1024
NB1 = B // RB1
RB2 = 1024
NB2 = B // RB2

F8S = 16.0
F8 = jnp.float8_e4m3fn


def _stats_body(feat_ref, lblp_ref, mem_ref, src_ref,
                featn_ref, memot_ref, lltot_ref,
                sums_ref, pres_ref, xprev_ref):
    i = pl.program_id(0)

    lblp = lblp_ref[0, 0, :]
    cls = lax.broadcasted_iota(jnp.int32, (CP, RB1), 0)
    eq = cls == lblp[None, :]
    d = lax.dot_general(
        eq.astype(F8), xprev_ref[...],
        (((1,), (0,)), ((), ())), preferred_element_type=jnp.float32)
    pm = jnp.any(eq, axis=1, keepdims=True).astype(jnp.float32)
    sums_ref[...] = jnp.where(i > 0, sums_ref[...] + d, 0.0)
    pres_ref[...] = jnp.where(i > 0, jnp.maximum(pres_ref[...], pm), 0.0)

    x = feat_ref[...]
    ss = jnp.sum(x * x, axis=1, keepdims=True)
    inv = F8S / jnp.maximum(jnp.sqrt(ss), 1e-12)
    xf8 = (x * inv).astype(F8)
    featn_ref[...] = xf8
    xprev_ref[...] = xf8

    @pl.when(i == NB1)
    def _():
        sums = sums_ref[0:C, :]
        present = pres_ref[0:C, :] > 0.0
        snorm = jnp.sqrt(jnp.sum(sums * sums, axis=1, keepdims=True))
        bc = jnp.where(present, sums / jnp.maximum(snorm, 1e-12), 0.0)
        mem = mem_ref[...]
        uw = jnp.sum(mem * bc, axis=1, keepdims=True)
        uw = 1.0 - (1.0 - uw) * present.astype(jnp.float32)
        nm = uw * mem + (1.0 - uw) * bc
        nnorm = jnp.sqrt(jnp.sum(nm * nm, axis=1, keepdims=True))
        nm = nm / jnp.maximum(nnorm, 1e-12)
        zpad = jnp.zeros((CP - C, D), jnp.float32)
        nmp = jnp.concatenate([nm * F8S, zpad], axis=0)
        srcp = jnp.concatenate([src_ref[...] * F8S, zpad], axis=0)
        memot_ref[:, 0:CP] = jnp.transpose(nmp).astype(F8)
        memot_ref[:, CP:M] = jnp.transpose(srcp).astype(F8)
        lltot_ref[...] = (jnp.sum(sums * nm) / F8S).reshape(1, 1)


def _loss_body(featn_ref, memot_ref, lltot_ref, out_ref, acc_ref):
    i = pl.program_id(0)
    raw = lax.dot_general(
        featn_ref[...], memot_ref[...],
        (((1,), (0,)), ((), ())), preferred_element_type=jnp.float32)
    e = jnp.exp2((raw * (1.4426950408889634 / (F8S * F8S))
                  ).astype(jnp.bfloat16))
    es = jnp.sum(e, axis=1, keepdims=True).astype(jnp.float32)
    lse = jnp.log(es - float(NPAD))
    acc_ref[...] = jnp.where(i > 0, acc_ref[...] + lse, lse)

    @pl.when(i == NB2 - 1)
    def _():
        out_ref[...] = (jnp.sum(acc_ref[...]).reshape(1, 1)
                        - lltot_ref[...]) / float(B)


@jax.jit
def kernel(feat, label, memory, source_memo):
    lbl3 = label.astype(jnp.int32).reshape(NB1, 1, RB1)

    featn, memot, lltot = pl.pallas_call(
        _stats_body,
        grid=(NB1 + 1,),
        in_specs=[
            pl.BlockSpec((RB1, D), lambda i: (jnp.minimum(i, NB1 - 1), 0)),
            pl.BlockSpec((1, 1, RB1), lambda i: (jnp.maximum(i - 1, 0), 0, 0)),
            pl.BlockSpec((C, D), lambda i: (0, 0)),
            pl.BlockSpec((C, D), lambda i: (0, 0)),
        ],
        out_specs=[
            pl.BlockSpec((RB1, D), lambda i: (jnp.minimum(i, NB1 - 1), 0)),
            pl.BlockSpec((D, M), lambda i: (0, 0)),
            pl.BlockSpec((1, 1), lambda i: (0, 0)),
        ],
        out_shape=[
            jax.ShapeDtypeStruct((B, D), F8),
            jax.ShapeDtypeStruct((D, M), F8),
            jax.ShapeDtypeStruct((1, 1), jnp.float32),
        ],
        scratch_shapes=[
            pltpu.VMEM((CP, D), jnp.float32),
            pltpu.VMEM((CP, 1), jnp.float32),
            pltpu.VMEM((RB1, D), F8),
        ],
        compiler_params=pltpu.CompilerParams(
            dimension_semantics=("arbitrary",)),
    )(feat, lbl3, memory, source_memo)

    loss2d = pl.pallas_call(
        _loss_body,
        grid=(NB2,),
        in_specs=[
            pl.BlockSpec((RB2, D), lambda i: (i, 0)),
            pl.BlockSpec((D, M), lambda i: (0, 0)),
            pl.BlockSpec((1, 1), lambda i: (0, 0)),
        ],
        out_specs=pl.BlockSpec((1, 1), lambda i: (0, 0)),
        out_shape=jax.ShapeDtypeStruct((1, 1), jnp.float32),
        scratch_shapes=[
            pltpu.VMEM((RB2, 1), jnp.float32),
        ],
        compiler_params=pltpu.CompilerParams(
            dimension_semantics=("arbitrary",)),
    )(featn, memot, lltot)

    return loss2d[0, 0]
